# baseline (speedup 1.0000x reference)
import functools

import jax
import jax.numpy as jnp
from jax import lax
from jax.experimental import pallas as pl
from jax.experimental.pallas import tpu as pltpu
from jax.experimental.pallas import tpu_sc as plsc

C = 384
B, H, W = 16, 56, 56
HB = 14
GRID_H = H // HB
L = 16
NCHUNK = C // L
PAD = 8
FC = 384.0


def _bfly_min(v, iota):
    for sh in (1, 2, 4, 8):
        v = jnp.minimum(v, jnp.take(v, (iota + sh) % L))
    return v


def _bfly_max(v, iota):
    for sh in (1, 2, 4, 8):
        v = jnp.maximum(v, jnp.take(v, (iota + sh) % L))
    return v


def _edge(ci, mn, mx):
    s = ci.astype(jnp.float32) / FC
    return mn * (1.0 - s) + mx * s


def _sc_mask_body(w_hbm, mask_hbm, wrow_v, bins_v, hist2_v, m_v, unit_v, shared_v):
    s = lax.axis_index("s")
    row = lax.axis_index("c")
    iota = lax.iota(jnp.int32, L)
    k1 = s
    k2 = jnp.where(s < 8, s + 16, s)

    pltpu.sync_copy(w_hbm.at[row], wrow_v)
    for j in range(NCHUNK):
        wrow_v[pl.ds(j * L, L)] = jnp.abs(wrow_v[pl.ds(j * L, L)])

    vmn = wrow_v[pl.ds(0, L)]
    vmx = vmn
    for j in range(1, NCHUNK):
        av = wrow_v[pl.ds(j * L, L)]
        vmn = jnp.minimum(vmn, av)
        vmx = jnp.maximum(vmx, av)
    mn = _bfly_min(vmn, iota)
    mx = _bfly_max(vmx, iota)
    scale = FC / (mx - mn)

    for j in range(NCHUNK):
        av = wrow_v[pl.ds(j * L, L)]
        c0 = ((av - mn) * scale).astype(jnp.int32)
        c0 = jnp.maximum(jnp.minimum(c0, C), 0)
        for _ in range(4):
            c0 = c0 - jnp.where(_edge(c0, mn, mx) > av, 1, 0)
            cnxt = jnp.minimum(c0 + 1, C)
            c0 = c0 + jnp.where((c0 < C) & (_edge(cnxt, mn, mx) <= av), 1, 0)
        bins_v[pl.ds(j * L, L)] = jnp.minimum(c0, C - 1)

    for t, k in ((0, k1), (1, k2)):
        gidx = iota + k * L
        acc = jnp.zeros((L,), jnp.float32)
        for j in range(NCHUNK):
            bj = bins_v[pl.ds(j * L, L)]
            for r in range(L):
                rb = jnp.take(bj, (iota + r) % L)
                acc = acc + jnp.where(rb == gidx, 1.0, 0.0)
        unit_v[t] = acc
        pltpu.sync_copy(unit_v.at[t], shared_v.at[k + PAD])
    plsc.subcore_barrier()

    @pl.when(s == 0)
    def _():
        pltpu.sync_copy(shared_v.at[pl.ds(PAD, NCHUNK)], hist2_v)
        rot1 = (iota + 1) % L
        rot2 = (iota + 2) % L
        best = jnp.full((L,), 9999, jnp.int32)
        for k in range(NCHUNK):
            h0 = hist2_v[k]
            if k + 1 < NCHUNK:
                hn = hist2_v[k + 1]
            else:
                hn = jnp.zeros((L,), jnp.float32)
            h1 = jnp.where(iota < L - 1, jnp.take(h0, rot1), jnp.take(hn, rot1))
            h2 = jnp.where(iota < L - 2, jnp.take(h0, rot2), jnp.take(hn, rot2))
            d0 = h1 - h0
            d1 = h2 - h1
            gidx = iota + k * L
            cond = (d0 <= 0.0) & (d1 > 0.0) & (gidx <= C - 3)
            best = jnp.minimum(best, jnp.where(cond, gidx, 9999))
        i_min = _bfly_min(best, iota)
        i_star = jnp.where(i_min > C - 3, 0, i_min).astype(jnp.float32)
        thr = mn + ((i_star + 2.0) * (mx - mn)) / FC
        for j in range(NCHUNK):
            av = wrow_v[pl.ds(j * L, L)]
            m_v[pl.ds(j * L, L)] = jnp.where(av >= thr, 1.0, 0.0)
        pltpu.sync_copy(m_v, mask_hbm.at[row])


def _exchange_kernel(m_ref, x0_ref, x1_ref, o1_ref, o2_ref):
    m1 = m_ref[0:1, :][:, None, None, :] != 0.0
    m2 = m_ref[1:2, :][:, None, None, :] != 0.0
    x0 = x0_ref[...]
    x1 = x1_ref[...]
    o1_ref[...] = jnp.where(m1, x0, x1)
    o2_ref[...] = jnp.where(m2, x1, x0)


def kernel(x0, x1, bn1_weight, bn2_weight):
    wstack = jnp.stack([bn1_weight, bn2_weight])
    sc_mask = functools.partial(
        pl.kernel,
        mesh=plsc.VectorSubcoreMesh(core_axis_name="c", subcore_axis_name="s"),
        out_type=jax.ShapeDtypeStruct((2, C), jnp.float32),
        scratch_types=[
            pltpu.VMEM((C,), jnp.float32),
            pltpu.VMEM((C,), jnp.int32),
            pltpu.VMEM((NCHUNK, L), jnp.float32),
            pltpu.VMEM((C,), jnp.float32),
            pltpu.VMEM((2, L), jnp.float32),
            pltpu.VMEM_SHARED((NCHUNK + PAD, L), jnp.float32),
        ],
    )(_sc_mask_body)
    masks = sc_mask(wstack)

    x0t = jnp.transpose(x0, (0, 2, 3, 1))
    x1t = jnp.transpose(x1, (0, 2, 3, 1))

    x_spec = pl.BlockSpec((1, HB, W, C), lambda b, h: (b, h, 0, 0))
    mask_spec = pl.BlockSpec((2, C), lambda b, h: (0, 0))
    out1, out2 = pl.pallas_call(
        _exchange_kernel,
        grid=(B, GRID_H),
        in_specs=[mask_spec, x_spec, x_spec],
        out_specs=[x_spec, x_spec],
        out_shape=[
            jax.ShapeDtypeStruct((B, H, W, C), jnp.float32),
            jax.ShapeDtypeStruct((B, H, W, C), jnp.float32),
        ],
    )(masks, x0t, x1t)
    return (
        jnp.transpose(out1, (0, 3, 1, 2)),
        jnp.transpose(out2, (0, 3, 1, 2)),
    )


# speedup vs baseline: 1.0594x; 1.0594x over previous
import functools

import jax
import jax.numpy as jnp
from jax import lax
from jax.experimental import pallas as pl
from jax.experimental.pallas import tpu as pltpu
from jax.experimental.pallas import tpu_sc as plsc

C = 384
B, H, W = 16, 56, 56
HB = 56
GRID_H = H // HB
L = 16
NCHUNK = C // L
PAD = 8
FC = 384.0


def _bfly_min(v, iota):
    for sh in (1, 2, 4, 8):
        v = jnp.minimum(v, jnp.take(v, (iota + sh) % L))
    return v


def _bfly_max(v, iota):
    for sh in (1, 2, 4, 8):
        v = jnp.maximum(v, jnp.take(v, (iota + sh) % L))
    return v


def _edge(ci, mn, mx):
    s = ci.astype(jnp.float32) / FC
    return mn * (1.0 - s) + mx * s


def _sc_mask_body(w_hbm, mask_hbm, wrow_v, bins_v, hist2_v, m_v, unit_v, shared_v):
    s = lax.axis_index("s")
    row = lax.axis_index("c")
    iota = lax.iota(jnp.int32, L)
    k1 = s
    k2 = jnp.where(s < 8, s + 16, s)

    pltpu.sync_copy(w_hbm.at[row], wrow_v)
    for j in range(NCHUNK):
        wrow_v[pl.ds(j * L, L)] = jnp.abs(wrow_v[pl.ds(j * L, L)])

    vmn = wrow_v[pl.ds(0, L)]
    vmx = vmn
    for j in range(1, NCHUNK):
        av = wrow_v[pl.ds(j * L, L)]
        vmn = jnp.minimum(vmn, av)
        vmx = jnp.maximum(vmx, av)
    mn = _bfly_min(vmn, iota)
    mx = _bfly_max(vmx, iota)
    scale = FC / (mx - mn)

    for j in range(NCHUNK):
        av = wrow_v[pl.ds(j * L, L)]
        c0 = ((av - mn) * scale).astype(jnp.int32)
        c0 = jnp.maximum(jnp.minimum(c0, C), 0)
        for _ in range(4):
            c0 = c0 - jnp.where(_edge(c0, mn, mx) > av, 1, 0)
            cnxt = jnp.minimum(c0 + 1, C)
            c0 = c0 + jnp.where((c0 < C) & (_edge(cnxt, mn, mx) <= av), 1, 0)
        bins_v[pl.ds(j * L, L)] = jnp.minimum(c0, C - 1)

    for t, k in ((0, k1), (1, k2)):
        gidx = iota + k * L
        acc = jnp.zeros((L,), jnp.float32)
        for j in range(NCHUNK):
            bj = bins_v[pl.ds(j * L, L)]
            for r in range(L):
                rb = jnp.take(bj, (iota + r) % L)
                acc = acc + jnp.where(rb == gidx, 1.0, 0.0)
        unit_v[t] = acc
        pltpu.sync_copy(unit_v.at[t], shared_v.at[k + PAD])
    plsc.subcore_barrier()

    @pl.when(s == 0)
    def _():
        pltpu.sync_copy(shared_v.at[pl.ds(PAD, NCHUNK)], hist2_v)
        rot1 = (iota + 1) % L
        rot2 = (iota + 2) % L
        best = jnp.full((L,), 9999, jnp.int32)
        for k in range(NCHUNK):
            h0 = hist2_v[k]
            if k + 1 < NCHUNK:
                hn = hist2_v[k + 1]
            else:
                hn = jnp.zeros((L,), jnp.float32)
            h1 = jnp.where(iota < L - 1, jnp.take(h0, rot1), jnp.take(hn, rot1))
            h2 = jnp.where(iota < L - 2, jnp.take(h0, rot2), jnp.take(hn, rot2))
            d0 = h1 - h0
            d1 = h2 - h1
            gidx = iota + k * L
            cond = (d0 <= 0.0) & (d1 > 0.0) & (gidx <= C - 3)
            best = jnp.minimum(best, jnp.where(cond, gidx, 9999))
        i_min = _bfly_min(best, iota)
        i_star = jnp.where(i_min > C - 3, 0, i_min).astype(jnp.float32)
        thr = mn + ((i_star + 2.0) * (mx - mn)) / FC
        for j in range(NCHUNK):
            av = wrow_v[pl.ds(j * L, L)]
            m_v[pl.ds(j * L, L)] = jnp.where(av >= thr, 1.0, 0.0)
        pltpu.sync_copy(m_v, mask_hbm.at[row])


def _exchange_kernel(m_ref, x0_ref, x1_ref, o1_ref, o2_ref):
    m1 = m_ref[0:1, :][:, None, None, :] != 0.0
    m2 = m_ref[1:2, :][:, None, None, :] != 0.0
    x0 = x0_ref[...]
    x1 = x1_ref[...]
    o1_ref[...] = jnp.where(m1, x0, x1)
    o2_ref[...] = jnp.where(m2, x1, x0)


def kernel(x0, x1, bn1_weight, bn2_weight):
    wstack = jnp.stack([bn1_weight, bn2_weight])
    sc_mask = functools.partial(
        pl.kernel,
        mesh=plsc.VectorSubcoreMesh(core_axis_name="c", subcore_axis_name="s"),
        out_type=jax.ShapeDtypeStruct((2, C), jnp.float32),
        scratch_types=[
            pltpu.VMEM((C,), jnp.float32),
            pltpu.VMEM((C,), jnp.int32),
            pltpu.VMEM((NCHUNK, L), jnp.float32),
            pltpu.VMEM((C,), jnp.float32),
            pltpu.VMEM((2, L), jnp.float32),
            pltpu.VMEM_SHARED((NCHUNK + PAD, L), jnp.float32),
        ],
    )(_sc_mask_body)
    masks = sc_mask(wstack)

    x0t = jnp.transpose(x0, (0, 2, 3, 1))
    x1t = jnp.transpose(x1, (0, 2, 3, 1))

    x_spec = pl.BlockSpec((1, HB, W, C), lambda b, h: (b, h, 0, 0))
    mask_spec = pl.BlockSpec((2, C), lambda b, h: (0, 0))
    out1, out2 = pl.pallas_call(
        _exchange_kernel,
        grid=(B, GRID_H),
        in_specs=[mask_spec, x_spec, x_spec],
        out_specs=[x_spec, x_spec],
        out_shape=[
            jax.ShapeDtypeStruct((B, H, W, C), jnp.float32),
            jax.ShapeDtypeStruct((B, H, W, C), jnp.float32),
        ],
    )(masks, x0t, x1t)
    return (
        jnp.transpose(out1, (0, 3, 1, 2)),
        jnp.transpose(out2, (0, 3, 1, 2)),
    )
